# baseline (speedup 1.0000x reference)
import functools

import jax
import jax.numpy as jnp
import numpy as np
from jax import lax
from jax.experimental import pallas as pl
from jax.experimental.pallas import tpu as pltpu
from jax.experimental.pallas import tpu_sc as plsc

N_NODES = 10000
N_EDGES = 320000
HIDDEN = 40
TOWERS = 5
F_IN = HIDDEN
F_OUT = HIDDEN // TOWERS
NUM_LAYERS = 2

_deg_hist = np.zeros(33, dtype=np.float64)
_deg_hist[32] = N_NODES
_bins = np.arange(33, dtype=np.float64)
AVG_DEG_LOG = float((np.log(_bins + 1.0) * _deg_hist).sum() / _deg_hist.sum())

D5 = TOWERS * F_IN
DP = 256
NCH = DP // 16
NPB = 80
NB = N_NODES // NPB
NPASS = 4
BLK = 64
ACC_ROWS = NPB + 1
ACC_SZ = ACC_ROWS * DP
OUT_B = NPB * DP
EPAD = N_EDGES + NB * BLK


def _hdot(a, b):
    return jnp.dot(a, b, precision=jax.lax.Precision.HIGHEST)


def _edge_pass(bmat, cmat, srcp, cperm, dloc, boff):
    mesh = plsc.VectorSubcoreMesh(core_axis_name="c", subcore_axis_name="s",
                                  num_cores=2, num_subcores=16)
    acc_t = jax.ShapeDtypeStruct((N_NODES * DP,), jnp.float32)

    @functools.partial(
        pl.kernel,
        compiler_params=pltpu.CompilerParams(needs_layout_passes=False),
        out_type=[acc_t, acc_t, acc_t, acc_t],
        mesh=mesh,
        scratch_types=[
            pltpu.VMEM((ACC_SZ,), jnp.float32),
            pltpu.VMEM((ACC_SZ,), jnp.float32),
            pltpu.VMEM((ACC_SZ,), jnp.float32),
            pltpu.VMEM((ACC_SZ,), jnp.float32),
            pltpu.VMEM((BLK, DP), jnp.float32),
            pltpu.VMEM((BLK, DP), jnp.float32),
            pltpu.VMEM((BLK,), jnp.int32),
            pltpu.VMEM((BLK,), jnp.int32),
            pltpu.VMEM((BLK,), jnp.int32),
            pltpu.VMEM((136,), jnp.int32),
            pltpu.SemaphoreType.DMA,
            pltpu.SemaphoreType.DMA,
        ],
    )
    def k(b_hbm, c_hbm, sp_hbm, cp_hbm, dl_hbm, bo_hbm,
          osum, osq, omin, omax,
          asum, asq, amin, amax, bbuf, cbuf, sidx, cidx, dbuf, boffb,
          sem1, sem2):
        wid = lax.axis_index("s") * 2 + lax.axis_index("c")
        pltpu.sync_copy(bo_hbm, boffb)
        iota16 = lax.iota(jnp.int32, 16)

        for p in range(NPASS):
            bucket = p * 32 + wid

            @pl.when(bucket < NB)
            def _():
                bvec = jnp.full((16,), bucket, jnp.int32)
                cur = jnp.max(plsc.load_gather(boffb, [bvec]))
                nxt = jnp.max(plsc.load_gather(boffb, [bvec + 1]))

                def initbody(i, _):
                    sl = pl.ds(i * 16, 16)
                    asum[sl] = jnp.zeros((16,), jnp.float32)
                    asq[sl] = jnp.zeros((16,), jnp.float32)
                    amin[sl] = jnp.full((16,), 3e38, jnp.float32)
                    amax[sl] = jnp.full((16,), -3e38, jnp.float32)
                    return 0

                lax.fori_loop(0, ACC_SZ // 16, initbody, 0, unroll=2)

                def blkbody(blk, _):
                    base = (cur + blk) * BLK
                    pltpu.sync_copy(sp_hbm.at[pl.ds(base, BLK)], sidx)
                    pltpu.sync_copy(cp_hbm.at[pl.ds(base, BLK)], cidx)
                    pltpu.sync_copy(dl_hbm.at[pl.ds(base, BLK)], dbuf)
                    d1 = pltpu.async_copy(b_hbm.at[sidx], bbuf, sem1)
                    d2 = pltpu.async_copy(c_hbm.at[cidx], cbuf, sem2)
                    d1.wait()
                    d2.wait()

                    def ebody(j, _):
                        row = plsc.load_gather(dbuf, [jnp.full((16,), j, jnp.int32)])
                        rbase = row * DP
                        for c in range(NCH):
                            addr = rbase + (c * 16) + iota16
                            g = bbuf[j, pl.ds(c * 16, 16)] + cbuf[j, pl.ds(c * 16, 16)]
                            plsc.addupdate_scatter(asum, [addr], g)
                            plsc.addupdate_scatter(asq, [addr], g * g)
                            mn = plsc.load_gather(amin, [addr])
                            plsc.store_scatter(amin, [addr], jnp.minimum(mn, g))
                            mx = plsc.load_gather(amax, [addr])
                            plsc.store_scatter(amax, [addr], jnp.maximum(mx, g))
                        return 0

                    lax.fori_loop(0, BLK, ebody, 0)
                    return 0

                lax.fori_loop(0, nxt - cur, blkbody, 0)

                ob = bucket * OUT_B
                pltpu.sync_copy(asum.at[pl.ds(0, OUT_B)], osum.at[pl.ds(ob, OUT_B)])
                pltpu.sync_copy(asq.at[pl.ds(0, OUT_B)], osq.at[pl.ds(ob, OUT_B)])
                pltpu.sync_copy(amin.at[pl.ds(0, OUT_B)], omin.at[pl.ds(ob, OUT_B)])
                pltpu.sync_copy(amax.at[pl.ds(0, OUT_B)], omax.at[pl.ds(ob, OUT_B)])

    return k(bmat, cmat, srcp, cperm, dloc, boff)


def _build_buckets(src, dst):
    bucket = dst // NPB
    order = jnp.argsort(bucket)
    sortedb = bucket[order]
    bounds = jnp.searchsorted(sortedb, jnp.arange(NB + 1, dtype=jnp.int32))
    starts = bounds[:-1]
    cnts = bounds[1:] - bounds[:-1]
    blocks = (cnts + (BLK - 1)) // BLK
    boff = jnp.concatenate([jnp.zeros((1,), jnp.int32), jnp.cumsum(blocks).astype(jnp.int32)])
    boff_pad = jnp.concatenate(
        [boff, jnp.full((136 - (NB + 1),), boff[-1], jnp.int32)])
    s = jnp.arange(EPAD, dtype=jnp.int32)
    bblk = (jnp.searchsorted(boff * BLK, s, side='right') - 1).astype(jnp.int32)
    bblk = jnp.clip(bblk, 0, NB - 1)
    r = s - boff[bblk] * BLK
    valid = r < cnts[bblk]
    eidx = order[jnp.clip(starts[bblk] + r, 0, N_EDGES - 1)].astype(jnp.int32)
    srcp = jnp.where(valid, src[eidx], 0).astype(jnp.int32)
    cperm = jnp.where(valid, eidx, 0).astype(jnp.int32)
    dloc = jnp.where(valid, dst[eidx] - bblk * NPB, NPB).astype(jnp.int32)
    return srcp, cperm, dloc, boff_pad


def _mlp_body(h_ref, w0, b0, w1, b1, w2, b2, o_ref):
    h = h_ref[...]
    o = jax.nn.relu(_hdot(h, w0[...]) + b0[...])
    o = jax.nn.relu(_hdot(o, w1[...]) + b1[...])
    o_ref[...] = _hdot(o, w2[...]) + b2[...]


def _mlp_pallas(h, mlp):
    (w0, b0), (w1, b1), (w2, b2) = mlp
    n = h.shape[0]
    blk = 2000
    grid = (n // blk,)
    return pl.pallas_call(
        _mlp_body,
        grid=grid,
        in_specs=[
            pl.BlockSpec((blk, h.shape[1]), lambda i: (i, 0)),
            pl.BlockSpec(w0.shape, lambda i: (0, 0)),
            pl.BlockSpec(b0.shape, lambda i: (0,)),
            pl.BlockSpec(w1.shape, lambda i: (0, 0)),
            pl.BlockSpec(b1.shape, lambda i: (0,)),
            pl.BlockSpec(w2.shape, lambda i: (0, 0)),
            pl.BlockSpec(b2.shape, lambda i: (0,)),
        ],
        out_specs=pl.BlockSpec((blk, w2.shape[1]), lambda i: (i, 0)),
        out_shape=jax.ShapeDtypeStruct((n, w2.shape[1]), jnp.float32),
    )(h, w0, b0, w1, b1, w2, b2)


def _layer_weights(lp):
    wd = jnp.concatenate([lp['pre'][t][0][0:F_IN] for t in range(TOWERS)], axis=1)
    ws = jnp.concatenate([lp['pre'][t][0][F_IN:2 * F_IN] for t in range(TOWERS)], axis=1)
    we = jnp.concatenate([lp['pre'][t][0][2 * F_IN:3 * F_IN] for t in range(TOWERS)], axis=1)
    bt = jnp.concatenate([lp['pre'][t][1] for t in range(TOWERS)], axis=0)
    wenc, benc = lp['edge_enc']
    wc = wenc @ we
    bc = benc @ we
    post_w = jnp.stack([lp['post'][t][0] for t in range(TOWERS)], axis=0)
    post_b = jnp.concatenate([lp['post'][t][1] for t in range(TOWERS)], axis=0)
    return wd, ws, bt, wc, bc, post_w, post_b


def _pna_layer(h, ea, perm_data, lp):
    srcp, cperm, dloc, boff = perm_data
    wd, ws, bt, wc, bc, post_w, post_b = _layer_weights(lp)
    n = h.shape[0]
    e = ea.shape[0]
    a = _hdot(h, wd) + bt
    bmat = jnp.concatenate(
        [_hdot(h, ws), jnp.zeros((n, DP - D5), jnp.float32)], axis=1)
    cmat = jnp.concatenate(
        [_hdot(ea, wc) + bc,
         jnp.ones((e, 1), jnp.float32),
         jnp.zeros((e, DP - D5 - 1), jnp.float32)], axis=1)

    gs, gq, gmn, gmx = _edge_pass(bmat, cmat, srcp, cperm, dloc, boff)
    gs = gs.reshape(n, DP)
    gq = gq.reshape(n, DP)[:, :D5]
    gmn = gmn.reshape(n, DP)[:, :D5]
    gmx = gmx.reshape(n, DP)[:, :D5]
    cnt = gs[:, D5]
    gsum = gs[:, :D5]

    cnt_c = jnp.maximum(cnt, 1.0)[:, None]
    s = cnt[:, None] * a + gsum
    mean = s / cnt_c
    sumsq = cnt[:, None] * (a * a) + 2.0 * a * gsum + gq
    mean2 = sumsq / cnt_c
    var = jax.nn.relu(mean2 - mean * mean)
    std = jnp.sqrt(var + 1e-5)
    has = (cnt > 0)[:, None]
    mn = jnp.where(has, a + gmn, 0.0)
    mx = jnp.where(has, a + gmx, 0.0)

    def t5(v):
        return v.reshape(n, TOWERS, F_IN)

    aggr = jnp.concatenate([t5(mean), t5(mn), t5(mx), t5(std)], axis=-1)
    logd = jnp.log(jnp.maximum(cnt, 1.0) + 1.0)[:, None, None]
    amp = aggr * (logd / AVG_DEG_LOG)
    att = aggr * (AVG_DEG_LOG / logd)
    x_t = jnp.broadcast_to(h[:, None, :], (n, TOWERS, F_IN))
    out = jnp.concatenate([x_t, aggr, amp, att], axis=-1)
    outs = jnp.einsum('ntf,tfo->nto', out, post_w,
                      precision=jax.lax.Precision.HIGHEST).reshape(n, HIDDEN) + post_b
    lw, lb = lp['lin']
    return _hdot(outs, lw) + lb


def _batch_norm(x, gb):
    mu = x.mean(axis=0)
    var = ((x - mu) ** 2).mean(axis=0)
    return gb[0] * (x - mu) / jnp.sqrt(var + 1e-5) + gb[1]


def kernel(x, edge_index, edge_attr, params):
    with jax.default_matmul_precision('highest'):
        return _kernel_impl(x, edge_index, edge_attr, params)


def _kernel_impl(x, edge_index, edge_attr, params):
    src = edge_index[0].astype(jnp.int32)
    dst = edge_index[1].astype(jnp.int32)
    perm_data = _build_buckets(src, dst)
    h = x @ params['node_emb'][0] + params['node_emb'][1]
    ea = edge_attr @ params['edge_emb'][0] + params['edge_emb'][1]
    xs_sum = h
    n_xs = 1
    for i in range(NUM_LAYERS):
        lp = params['layers'][i]
        c = _pna_layer(h, ea, perm_data, lp)
        c = _batch_norm(c, lp['bn'])
        c = jax.nn.relu(c)
        xs_sum = xs_sum + c
        n_xs += 1
        h = xs_sum / n_xs
        if i + 1 < NUM_LAYERS:
            (u_w, u_b), (v_w, v_b) = lp['emlp']
            u1 = u_w[0:HIDDEN]
            u2 = u_w[HIDDEN:2 * HIDDEN]
            u3 = u_w[2 * HIDDEN:3 * HIDDEN]
            upd = jax.nn.relu(h[src] @ u1 + h[dst] @ u2 + ea @ u3 + u_b)
            ea = ea + (upd @ v_w + v_b) / 2.0
    return _mlp_pallas(h, params['mlp'])


# speedup vs baseline: 1.0307x; 1.0307x over previous
import functools

import jax
import jax.numpy as jnp
import numpy as np
from jax import lax
from jax.experimental import pallas as pl
from jax.experimental.pallas import tpu as pltpu
from jax.experimental.pallas import tpu_sc as plsc

N_NODES = 10000
N_EDGES = 320000
HIDDEN = 40
TOWERS = 5
F_IN = HIDDEN
F_OUT = HIDDEN // TOWERS
NUM_LAYERS = 2

_deg_hist = np.zeros(33, dtype=np.float64)
_deg_hist[32] = N_NODES
_bins = np.arange(33, dtype=np.float64)
AVG_DEG_LOG = float((np.log(_bins + 1.0) * _deg_hist).sum() / _deg_hist.sum())

D5 = TOWERS * F_IN
DP = 256
NCH = DP // 16
NCH_USED = (D5 + 1 + 15) // 16
NPB = 80
NB = N_NODES // NPB
NPASS = 4
BLK = 64
ACC_ROWS = NPB + 1
ACC_SZ = ACC_ROWS * DP
OUT_B = NPB * DP
EPAD = N_EDGES + NB * BLK


def _hdot(a, b):
    return jnp.dot(a, b, precision=jax.lax.Precision.HIGHEST)


def _edge_pass(bmat, cmat, srcp, cperm, dloc, boff):
    mesh = plsc.VectorSubcoreMesh(core_axis_name="c", subcore_axis_name="s",
                                  num_cores=2, num_subcores=16)
    acc_t = jax.ShapeDtypeStruct((N_NODES * DP,), jnp.float32)

    @functools.partial(
        pl.kernel,
        compiler_params=pltpu.CompilerParams(needs_layout_passes=False),
        out_type=[acc_t, acc_t, acc_t, acc_t],
        mesh=mesh,
        scratch_types=[
            pltpu.VMEM((ACC_SZ,), jnp.float32),
            pltpu.VMEM((ACC_SZ,), jnp.float32),
            pltpu.VMEM((ACC_SZ,), jnp.float32),
            pltpu.VMEM((ACC_SZ,), jnp.float32),
            pltpu.VMEM((BLK, DP), jnp.float32),
            pltpu.VMEM((BLK, DP), jnp.float32),
            pltpu.VMEM((BLK,), jnp.int32),
            pltpu.VMEM((BLK,), jnp.int32),
            pltpu.VMEM((BLK,), jnp.int32),
            pltpu.VMEM((136,), jnp.int32),
            pltpu.SemaphoreType.DMA,
            pltpu.SemaphoreType.DMA,
        ],
    )
    def k(b_hbm, c_hbm, sp_hbm, cp_hbm, dl_hbm, bo_hbm,
          osum, osq, omin, omax,
          asum, asq, amin, amax, bbuf, cbuf, sidx, cidx, dbuf, boffb,
          sem1, sem2):
        wid = lax.axis_index("s") * 2 + lax.axis_index("c")
        pltpu.sync_copy(bo_hbm, boffb)
        iota16 = lax.iota(jnp.int32, 16)

        for p in range(NPASS):
            bucket = p * 32 + wid

            @pl.when(bucket < NB)
            def _():
                bvec = jnp.full((16,), bucket, jnp.int32)
                cur = jnp.max(plsc.load_gather(boffb, [bvec]))
                nxt = jnp.max(plsc.load_gather(boffb, [bvec + 1]))

                def initbody(i, _):
                    sl = pl.ds(i * 16, 16)
                    asum[sl] = jnp.zeros((16,), jnp.float32)
                    asq[sl] = jnp.zeros((16,), jnp.float32)
                    amin[sl] = jnp.full((16,), 3e38, jnp.float32)
                    amax[sl] = jnp.full((16,), -3e38, jnp.float32)
                    return 0

                lax.fori_loop(0, ACC_SZ // 16, initbody, 0, unroll=2)

                def blkbody(blk, _):
                    base = (cur + blk) * BLK
                    pltpu.sync_copy(sp_hbm.at[pl.ds(base, BLK)], sidx)
                    pltpu.sync_copy(cp_hbm.at[pl.ds(base, BLK)], cidx)
                    pltpu.sync_copy(dl_hbm.at[pl.ds(base, BLK)], dbuf)
                    d1 = pltpu.async_copy(b_hbm.at[sidx], bbuf, sem1)
                    d2 = pltpu.async_copy(c_hbm.at[cidx], cbuf, sem2)
                    d1.wait()
                    d2.wait()

                    def ebody(j, _):
                        row = plsc.load_gather(dbuf, [jnp.full((16,), j, jnp.int32)])
                        rbase = row * DP
                        for c in range(NCH_USED):
                            addr = rbase + (c * 16) + iota16
                            g = bbuf[j, pl.ds(c * 16, 16)] + cbuf[j, pl.ds(c * 16, 16)]
                            plsc.addupdate_scatter(asum, [addr], g)
                            plsc.addupdate_scatter(asq, [addr], g * g)
                            mn = plsc.load_gather(amin, [addr])
                            plsc.store_scatter(amin, [addr], jnp.minimum(mn, g))
                            mx = plsc.load_gather(amax, [addr])
                            plsc.store_scatter(amax, [addr], jnp.maximum(mx, g))
                        return 0

                    lax.fori_loop(0, BLK, ebody, 0)
                    return 0

                lax.fori_loop(0, nxt - cur, blkbody, 0)

                ob = bucket * OUT_B
                pltpu.sync_copy(asum.at[pl.ds(0, OUT_B)], osum.at[pl.ds(ob, OUT_B)])
                pltpu.sync_copy(asq.at[pl.ds(0, OUT_B)], osq.at[pl.ds(ob, OUT_B)])
                pltpu.sync_copy(amin.at[pl.ds(0, OUT_B)], omin.at[pl.ds(ob, OUT_B)])
                pltpu.sync_copy(amax.at[pl.ds(0, OUT_B)], omax.at[pl.ds(ob, OUT_B)])

    return k(bmat, cmat, srcp, cperm, dloc, boff)


def _build_buckets(src, dst):
    bucket = dst // NPB
    order = jnp.argsort(bucket)
    sortedb = bucket[order]
    bounds = jnp.searchsorted(sortedb, jnp.arange(NB + 1, dtype=jnp.int32))
    starts = bounds[:-1]
    cnts = bounds[1:] - bounds[:-1]
    blocks = (cnts + (BLK - 1)) // BLK
    boff = jnp.concatenate([jnp.zeros((1,), jnp.int32), jnp.cumsum(blocks).astype(jnp.int32)])
    boff_pad = jnp.concatenate(
        [boff, jnp.full((136 - (NB + 1),), boff[-1], jnp.int32)])
    s = jnp.arange(EPAD, dtype=jnp.int32)
    bblk = (jnp.searchsorted(boff * BLK, s, side='right') - 1).astype(jnp.int32)
    bblk = jnp.clip(bblk, 0, NB - 1)
    r = s - boff[bblk] * BLK
    valid = r < cnts[bblk]
    eidx = order[jnp.clip(starts[bblk] + r, 0, N_EDGES - 1)].astype(jnp.int32)
    srcp = jnp.where(valid, src[eidx], 0).astype(jnp.int32)
    cperm = jnp.where(valid, eidx, 0).astype(jnp.int32)
    dloc = jnp.where(valid, dst[eidx] - bblk * NPB, NPB).astype(jnp.int32)
    return srcp, cperm, dloc, boff_pad


def _mlp_body(h_ref, w0, b0, w1, b1, w2, b2, o_ref):
    h = h_ref[...]
    o = jax.nn.relu(_hdot(h, w0[...]) + b0[...])
    o = jax.nn.relu(_hdot(o, w1[...]) + b1[...])
    o_ref[...] = _hdot(o, w2[...]) + b2[...]


def _mlp_pallas(h, mlp):
    (w0, b0), (w1, b1), (w2, b2) = mlp
    n = h.shape[0]
    blk = 2000
    grid = (n // blk,)
    return pl.pallas_call(
        _mlp_body,
        grid=grid,
        in_specs=[
            pl.BlockSpec((blk, h.shape[1]), lambda i: (i, 0)),
            pl.BlockSpec(w0.shape, lambda i: (0, 0)),
            pl.BlockSpec(b0.shape, lambda i: (0,)),
            pl.BlockSpec(w1.shape, lambda i: (0, 0)),
            pl.BlockSpec(b1.shape, lambda i: (0,)),
            pl.BlockSpec(w2.shape, lambda i: (0, 0)),
            pl.BlockSpec(b2.shape, lambda i: (0,)),
        ],
        out_specs=pl.BlockSpec((blk, w2.shape[1]), lambda i: (i, 0)),
        out_shape=jax.ShapeDtypeStruct((n, w2.shape[1]), jnp.float32),
    )(h, w0, b0, w1, b1, w2, b2)


def _layer_weights(lp):
    wd = jnp.concatenate([lp['pre'][t][0][0:F_IN] for t in range(TOWERS)], axis=1)
    ws = jnp.concatenate([lp['pre'][t][0][F_IN:2 * F_IN] for t in range(TOWERS)], axis=1)
    we = jnp.concatenate([lp['pre'][t][0][2 * F_IN:3 * F_IN] for t in range(TOWERS)], axis=1)
    bt = jnp.concatenate([lp['pre'][t][1] for t in range(TOWERS)], axis=0)
    wenc, benc = lp['edge_enc']
    wc = wenc @ we
    bc = benc @ we
    post_w = jnp.stack([lp['post'][t][0] for t in range(TOWERS)], axis=0)
    post_b = jnp.concatenate([lp['post'][t][1] for t in range(TOWERS)], axis=0)
    return wd, ws, bt, wc, bc, post_w, post_b


def _pna_layer(h, ea, perm_data, lp):
    srcp, cperm, dloc, boff = perm_data
    wd, ws, bt, wc, bc, post_w, post_b = _layer_weights(lp)
    n = h.shape[0]
    e = ea.shape[0]
    a = _hdot(h, wd) + bt
    bmat = jnp.concatenate(
        [_hdot(h, ws), jnp.zeros((n, DP - D5), jnp.float32)], axis=1)
    cmat = jnp.concatenate(
        [_hdot(ea, wc) + bc,
         jnp.ones((e, 1), jnp.float32),
         jnp.zeros((e, DP - D5 - 1), jnp.float32)], axis=1)

    gs, gq, gmn, gmx = _edge_pass(bmat, cmat, srcp, cperm, dloc, boff)
    gs = gs.reshape(n, DP)
    gq = gq.reshape(n, DP)[:, :D5]
    gmn = gmn.reshape(n, DP)[:, :D5]
    gmx = gmx.reshape(n, DP)[:, :D5]
    cnt = gs[:, D5]
    gsum = gs[:, :D5]

    cnt_c = jnp.maximum(cnt, 1.0)[:, None]
    s = cnt[:, None] * a + gsum
    mean = s / cnt_c
    sumsq = cnt[:, None] * (a * a) + 2.0 * a * gsum + gq
    mean2 = sumsq / cnt_c
    var = jax.nn.relu(mean2 - mean * mean)
    std = jnp.sqrt(var + 1e-5)
    has = (cnt > 0)[:, None]
    mn = jnp.where(has, a + gmn, 0.0)
    mx = jnp.where(has, a + gmx, 0.0)

    def t5(v):
        return v.reshape(n, TOWERS, F_IN)

    aggr = jnp.concatenate([t5(mean), t5(mn), t5(mx), t5(std)], axis=-1)
    logd = jnp.log(jnp.maximum(cnt, 1.0) + 1.0)[:, None, None]
    amp = aggr * (logd / AVG_DEG_LOG)
    att = aggr * (AVG_DEG_LOG / logd)
    x_t = jnp.broadcast_to(h[:, None, :], (n, TOWERS, F_IN))
    out = jnp.concatenate([x_t, aggr, amp, att], axis=-1)
    outs = jnp.einsum('ntf,tfo->nto', out, post_w,
                      precision=jax.lax.Precision.HIGHEST).reshape(n, HIDDEN) + post_b
    lw, lb = lp['lin']
    return _hdot(outs, lw) + lb


def _batch_norm(x, gb):
    mu = x.mean(axis=0)
    var = ((x - mu) ** 2).mean(axis=0)
    return gb[0] * (x - mu) / jnp.sqrt(var + 1e-5) + gb[1]


def kernel(x, edge_index, edge_attr, params):
    with jax.default_matmul_precision('highest'):
        return _kernel_impl(x, edge_index, edge_attr, params)


def _kernel_impl(x, edge_index, edge_attr, params):
    src = edge_index[0].astype(jnp.int32)
    dst = edge_index[1].astype(jnp.int32)
    perm_data = _build_buckets(src, dst)
    h = x @ params['node_emb'][0] + params['node_emb'][1]
    ea = edge_attr @ params['edge_emb'][0] + params['edge_emb'][1]
    xs_sum = h
    n_xs = 1
    for i in range(NUM_LAYERS):
        lp = params['layers'][i]
        c = _pna_layer(h, ea, perm_data, lp)
        c = _batch_norm(c, lp['bn'])
        c = jax.nn.relu(c)
        xs_sum = xs_sum + c
        n_xs += 1
        h = xs_sum / n_xs
        if i + 1 < NUM_LAYERS:
            (u_w, u_b), (v_w, v_b) = lp['emlp']
            u1 = u_w[0:HIDDEN]
            u2 = u_w[HIDDEN:2 * HIDDEN]
            u3 = u_w[2 * HIDDEN:3 * HIDDEN]
            upd = jax.nn.relu(h[src] @ u1 + h[dst] @ u2 + ea @ u3 + u_b)
            ea = ea + (upd @ v_w + v_b) / 2.0
    return _mlp_pallas(h, params['mlp'])


# speedup vs baseline: 1.0889x; 1.0565x over previous
import functools

import jax
import jax.numpy as jnp
import numpy as np
from jax import lax
from jax.experimental import pallas as pl
from jax.experimental.pallas import tpu as pltpu
from jax.experimental.pallas import tpu_sc as plsc

N_NODES = 10000
N_EDGES = 320000
HIDDEN = 40
TOWERS = 5
F_IN = HIDDEN
F_OUT = HIDDEN // TOWERS
NUM_LAYERS = 2

_deg_hist = np.zeros(33, dtype=np.float64)
_deg_hist[32] = N_NODES
_bins = np.arange(33, dtype=np.float64)
AVG_DEG_LOG = float((np.log(_bins + 1.0) * _deg_hist).sum() / _deg_hist.sum())

D5 = TOWERS * F_IN
DP = 256
NCH = DP // 16
NCH_USED = (D5 + 1 + 15) // 16
NPB = 80
NB = N_NODES // NPB
NPASS = 4
BLK = 64
ACC_ROWS = NPB + 1
ACC_SZ = ACC_ROWS * DP
OUT_B = NPB * DP
EPAD = N_EDGES + NB * BLK


def _hdot(a, b):
    return jnp.dot(a, b, precision=jax.lax.Precision.HIGHEST)


def _edge_pass(bmat, cmat, srcp, cperm, dloc, boff):
    mesh = plsc.VectorSubcoreMesh(core_axis_name="c", subcore_axis_name="s",
                                  num_cores=2, num_subcores=16)
    acc_t = jax.ShapeDtypeStruct((N_NODES * DP,), jnp.float32)

    @functools.partial(
        pl.kernel,
        compiler_params=pltpu.CompilerParams(needs_layout_passes=False),
        out_type=[acc_t, acc_t, acc_t, acc_t],
        mesh=mesh,
        scratch_types=[
            pltpu.VMEM((ACC_SZ,), jnp.float32),
            pltpu.VMEM((ACC_SZ,), jnp.float32),
            pltpu.VMEM((ACC_SZ,), jnp.float32),
            pltpu.VMEM((ACC_SZ,), jnp.float32),
            pltpu.VMEM((BLK, DP), jnp.float32),
            pltpu.VMEM((BLK, DP), jnp.float32),
            pltpu.VMEM((BLK,), jnp.int32),
            pltpu.VMEM((BLK,), jnp.int32),
            pltpu.VMEM((BLK,), jnp.int32),
            pltpu.VMEM((136,), jnp.int32),
            pltpu.SemaphoreType.DMA,
            pltpu.SemaphoreType.DMA,
        ],
    )
    def k(b_hbm, c_hbm, sp_hbm, cp_hbm, dl_hbm, bo_hbm,
          osum, osq, omin, omax,
          asum, asq, amin, amax, bbuf, cbuf, sidx, cidx, dbuf, boffb,
          sem1, sem2):
        wid = lax.axis_index("s") * 2 + lax.axis_index("c")
        pltpu.sync_copy(bo_hbm, boffb)
        iota16 = lax.iota(jnp.int32, 16)

        for p in range(NPASS):
            bucket = p * 32 + wid

            @pl.when(bucket < NB)
            def _():
                bvec = jnp.full((16,), bucket, jnp.int32)
                cur = jnp.max(plsc.load_gather(boffb, [bvec]))
                nxt = jnp.max(plsc.load_gather(boffb, [bvec + 1]))

                def initbody(i, _):
                    sl = pl.ds(i * 16, 16)
                    asum[sl] = jnp.zeros((16,), jnp.float32)
                    asq[sl] = jnp.zeros((16,), jnp.float32)
                    amin[sl] = jnp.full((16,), 3e38, jnp.float32)
                    amax[sl] = jnp.full((16,), -3e38, jnp.float32)
                    return 0

                lax.fori_loop(0, ACC_SZ // 16, initbody, 0, unroll=2)

                def blkbody(blk, _):
                    base = (cur + blk) * BLK
                    pltpu.sync_copy(sp_hbm.at[pl.ds(base, BLK)], sidx)
                    pltpu.sync_copy(cp_hbm.at[pl.ds(base, BLK)], cidx)
                    pltpu.sync_copy(dl_hbm.at[pl.ds(base, BLK)], dbuf)
                    d1 = pltpu.async_copy(b_hbm.at[sidx], bbuf, sem1)
                    d2 = pltpu.async_copy(c_hbm.at[cidx], cbuf, sem2)
                    d1.wait()
                    d2.wait()

                    def ebody(j, _):
                        row = jnp.max(
                            plsc.load_gather(dbuf, [jnp.full((16,), j, jnp.int32)]))
                        rbase = row * DP
                        for c in range(NCH_USED):
                            sl = pl.ds(rbase + c * 16, 16)
                            g = bbuf[j, pl.ds(c * 16, 16)] + cbuf[j, pl.ds(c * 16, 16)]
                            asum[sl] = asum[sl] + g
                            asq[sl] = asq[sl] + g * g
                            amin[sl] = jnp.minimum(amin[sl], g)
                            amax[sl] = jnp.maximum(amax[sl], g)
                        return 0

                    lax.fori_loop(0, BLK, ebody, 0)
                    return 0

                lax.fori_loop(0, nxt - cur, blkbody, 0)

                ob = bucket * OUT_B
                pltpu.sync_copy(asum.at[pl.ds(0, OUT_B)], osum.at[pl.ds(ob, OUT_B)])
                pltpu.sync_copy(asq.at[pl.ds(0, OUT_B)], osq.at[pl.ds(ob, OUT_B)])
                pltpu.sync_copy(amin.at[pl.ds(0, OUT_B)], omin.at[pl.ds(ob, OUT_B)])
                pltpu.sync_copy(amax.at[pl.ds(0, OUT_B)], omax.at[pl.ds(ob, OUT_B)])

    return k(bmat, cmat, srcp, cperm, dloc, boff)


def _build_buckets(src, dst):
    bucket = dst // NPB
    order = jnp.argsort(bucket)
    sortedb = bucket[order]
    bounds = jnp.searchsorted(sortedb, jnp.arange(NB + 1, dtype=jnp.int32))
    starts = bounds[:-1]
    cnts = bounds[1:] - bounds[:-1]
    blocks = (cnts + (BLK - 1)) // BLK
    boff = jnp.concatenate([jnp.zeros((1,), jnp.int32), jnp.cumsum(blocks).astype(jnp.int32)])
    boff_pad = jnp.concatenate(
        [boff, jnp.full((136 - (NB + 1),), boff[-1], jnp.int32)])
    s = jnp.arange(EPAD, dtype=jnp.int32)
    bblk = (jnp.searchsorted(boff * BLK, s, side='right') - 1).astype(jnp.int32)
    bblk = jnp.clip(bblk, 0, NB - 1)
    r = s - boff[bblk] * BLK
    valid = r < cnts[bblk]
    eidx = order[jnp.clip(starts[bblk] + r, 0, N_EDGES - 1)].astype(jnp.int32)
    srcp = jnp.where(valid, src[eidx], 0).astype(jnp.int32)
    cperm = jnp.where(valid, eidx, 0).astype(jnp.int32)
    dloc = jnp.where(valid, dst[eidx] - bblk * NPB, NPB).astype(jnp.int32)
    return srcp, cperm, dloc, boff_pad


def _mlp_body(h_ref, w0, b0, w1, b1, w2, b2, o_ref):
    h = h_ref[...]
    o = jax.nn.relu(_hdot(h, w0[...]) + b0[...])
    o = jax.nn.relu(_hdot(o, w1[...]) + b1[...])
    o_ref[...] = _hdot(o, w2[...]) + b2[...]


def _mlp_pallas(h, mlp):
    (w0, b0), (w1, b1), (w2, b2) = mlp
    n = h.shape[0]
    blk = 2000
    grid = (n // blk,)
    return pl.pallas_call(
        _mlp_body,
        grid=grid,
        in_specs=[
            pl.BlockSpec((blk, h.shape[1]), lambda i: (i, 0)),
            pl.BlockSpec(w0.shape, lambda i: (0, 0)),
            pl.BlockSpec(b0.shape, lambda i: (0,)),
            pl.BlockSpec(w1.shape, lambda i: (0, 0)),
            pl.BlockSpec(b1.shape, lambda i: (0,)),
            pl.BlockSpec(w2.shape, lambda i: (0, 0)),
            pl.BlockSpec(b2.shape, lambda i: (0,)),
        ],
        out_specs=pl.BlockSpec((blk, w2.shape[1]), lambda i: (i, 0)),
        out_shape=jax.ShapeDtypeStruct((n, w2.shape[1]), jnp.float32),
    )(h, w0, b0, w1, b1, w2, b2)


def _layer_weights(lp):
    wd = jnp.concatenate([lp['pre'][t][0][0:F_IN] for t in range(TOWERS)], axis=1)
    ws = jnp.concatenate([lp['pre'][t][0][F_IN:2 * F_IN] for t in range(TOWERS)], axis=1)
    we = jnp.concatenate([lp['pre'][t][0][2 * F_IN:3 * F_IN] for t in range(TOWERS)], axis=1)
    bt = jnp.concatenate([lp['pre'][t][1] for t in range(TOWERS)], axis=0)
    wenc, benc = lp['edge_enc']
    wc = wenc @ we
    bc = benc @ we
    post_w = jnp.stack([lp['post'][t][0] for t in range(TOWERS)], axis=0)
    post_b = jnp.concatenate([lp['post'][t][1] for t in range(TOWERS)], axis=0)
    return wd, ws, bt, wc, bc, post_w, post_b


def _pna_layer(h, ea, perm_data, lp):
    srcp, cperm, dloc, boff = perm_data
    wd, ws, bt, wc, bc, post_w, post_b = _layer_weights(lp)
    n = h.shape[0]
    e = ea.shape[0]
    a = _hdot(h, wd) + bt
    bmat = jnp.concatenate(
        [_hdot(h, ws), jnp.zeros((n, DP - D5), jnp.float32)], axis=1)
    cmat = jnp.concatenate(
        [_hdot(ea, wc) + bc,
         jnp.ones((e, 1), jnp.float32),
         jnp.zeros((e, DP - D5 - 1), jnp.float32)], axis=1)

    gs, gq, gmn, gmx = _edge_pass(bmat, cmat, srcp, cperm, dloc, boff)
    gs = gs.reshape(n, DP)
    gq = gq.reshape(n, DP)[:, :D5]
    gmn = gmn.reshape(n, DP)[:, :D5]
    gmx = gmx.reshape(n, DP)[:, :D5]
    cnt = gs[:, D5]
    gsum = gs[:, :D5]

    cnt_c = jnp.maximum(cnt, 1.0)[:, None]
    s = cnt[:, None] * a + gsum
    mean = s / cnt_c
    sumsq = cnt[:, None] * (a * a) + 2.0 * a * gsum + gq
    mean2 = sumsq / cnt_c
    var = jax.nn.relu(mean2 - mean * mean)
    std = jnp.sqrt(var + 1e-5)
    has = (cnt > 0)[:, None]
    mn = jnp.where(has, a + gmn, 0.0)
    mx = jnp.where(has, a + gmx, 0.0)

    def t5(v):
        return v.reshape(n, TOWERS, F_IN)

    aggr = jnp.concatenate([t5(mean), t5(mn), t5(mx), t5(std)], axis=-1)
    logd = jnp.log(jnp.maximum(cnt, 1.0) + 1.0)[:, None, None]
    amp = aggr * (logd / AVG_DEG_LOG)
    att = aggr * (AVG_DEG_LOG / logd)
    x_t = jnp.broadcast_to(h[:, None, :], (n, TOWERS, F_IN))
    out = jnp.concatenate([x_t, aggr, amp, att], axis=-1)
    outs = jnp.einsum('ntf,tfo->nto', out, post_w,
                      precision=jax.lax.Precision.HIGHEST).reshape(n, HIDDEN) + post_b
    lw, lb = lp['lin']
    return _hdot(outs, lw) + lb


def _batch_norm(x, gb):
    mu = x.mean(axis=0)
    var = ((x - mu) ** 2).mean(axis=0)
    return gb[0] * (x - mu) / jnp.sqrt(var + 1e-5) + gb[1]


def kernel(x, edge_index, edge_attr, params):
    with jax.default_matmul_precision('highest'):
        return _kernel_impl(x, edge_index, edge_attr, params)


def _kernel_impl(x, edge_index, edge_attr, params):
    src = edge_index[0].astype(jnp.int32)
    dst = edge_index[1].astype(jnp.int32)
    perm_data = _build_buckets(src, dst)
    h = x @ params['node_emb'][0] + params['node_emb'][1]
    ea = edge_attr @ params['edge_emb'][0] + params['edge_emb'][1]
    xs_sum = h
    n_xs = 1
    for i in range(NUM_LAYERS):
        lp = params['layers'][i]
        c = _pna_layer(h, ea, perm_data, lp)
        c = _batch_norm(c, lp['bn'])
        c = jax.nn.relu(c)
        xs_sum = xs_sum + c
        n_xs += 1
        h = xs_sum / n_xs
        if i + 1 < NUM_LAYERS:
            (u_w, u_b), (v_w, v_b) = lp['emlp']
            u1 = u_w[0:HIDDEN]
            u2 = u_w[HIDDEN:2 * HIDDEN]
            u3 = u_w[2 * HIDDEN:3 * HIDDEN]
            upd = jax.nn.relu(h[src] @ u1 + h[dst] @ u2 + ea @ u3 + u_b)
            ea = ea + (upd @ v_w + v_b) / 2.0
    return _mlp_pallas(h, params['mlp'])
